# Initial kernel scaffold; baseline (speedup 1.0000x reference)
import jax
import jax.numpy as jnp
from jax.experimental import pallas as pl


def kernel(x, router_w, gate_w, up_w, down_w, shared_gate_w, shared_up_w, shared_down_w):
    raise NotImplementedError("write your pallas kernel here")


# speedup vs baseline: 1.1299x; 1.1299x over previous
import jax
import jax.numpy as jnp
from jax.experimental import pallas as pl
from jax.experimental.pallas import tpu as pltpu

_E = 8
_TT = 256


def _moe_body(rw_ref, x_ref, gate_ref, up_ref, down_ref, out_ref, comb_ref):
    e = pl.program_id(1)

    @pl.when(e == 0)
    def _():
        xb = x_ref[...]
        logits = jnp.dot(xb, rw_ref[...], preferred_element_type=jnp.float32)
        m = jnp.max(logits, axis=-1, keepdims=True)
        ex = jnp.exp(logits - m)
        p = ex / jnp.sum(ex, axis=-1, keepdims=True)
        idx = jax.lax.broadcasted_iota(jnp.int32, p.shape, 1)
        m1 = jnp.max(p, axis=-1, keepdims=True)
        i1 = jnp.min(jnp.where(p == m1, idx, _E), axis=-1, keepdims=True)
        pm = jnp.where(idx == i1, -jnp.inf, p)
        m2 = jnp.max(pm, axis=-1, keepdims=True)
        i2 = jnp.min(jnp.where(pm == m2, idx, _E), axis=-1, keepdims=True)
        comb = jnp.where(idx == i1, m1, 0.0) + jnp.where(idx == i2, m2, 0.0)
        comb_ref[:, :_E] = comb
        comb_ref[:, _E:] = jnp.ones_like(comb_ref[:, _E:])

    xb = x_ref[...]
    g = jnp.dot(xb, gate_ref[0], preferred_element_type=jnp.float32)
    u = jnp.dot(xb, up_ref[0], preferred_element_type=jnp.float32)
    hmid = (g * jax.nn.sigmoid(g)) * u
    o = jnp.dot(hmid, down_ref[0], preferred_element_type=jnp.float32)
    comb = comb_ref[...]
    lane = jax.lax.broadcasted_iota(jnp.int32, comb.shape, 1)
    c = jnp.sum(jnp.where(lane == e, comb, 0.0), axis=1, keepdims=True)
    acc = c * o

    @pl.when(e == 0)
    def _():
        out_ref[...] = acc

    @pl.when(e > 0)
    def _():
        out_ref[...] += acc


def kernel(x, router_w, gate_w, up_w, down_w, shared_gate_w, shared_up_w,
           shared_down_w):
    b, s, h = x.shape
    d = gate_w.shape[2]
    T = b * s
    x_flat = x.reshape(T, h)

    gate9 = jnp.concatenate([gate_w, shared_gate_w[None]], axis=0)
    up9 = jnp.concatenate([up_w, shared_up_w[None]], axis=0)
    down9 = jnp.concatenate([down_w, shared_down_w[None]], axis=0)

    nT = T // _TT
    out = pl.pallas_call(
        _moe_body,
        grid=(nT, _E + 1),
        in_specs=[
            pl.BlockSpec((h, _E), lambda j, e: (0, 0)),
            pl.BlockSpec((_TT, h), lambda j, e: (j, 0)),
            pl.BlockSpec((1, h, d), lambda j, e: (e, 0, 0)),
            pl.BlockSpec((1, h, d), lambda j, e: (e, 0, 0)),
            pl.BlockSpec((1, d, h), lambda j, e: (e, 0, 0)),
        ],
        out_specs=pl.BlockSpec((_TT, h), lambda j, e: (j, 0)),
        out_shape=jax.ShapeDtypeStruct((T, h), jnp.float32),
        scratch_shapes=[pltpu.VMEM((_TT, 16), jnp.float32)],
    )(router_w, x_flat, gate9, up9, down9)

    return out.reshape(b, s, h)
